# baseline (speedup 1.0000x reference)
import functools

import jax
import jax.numpy as jnp
from jax import lax
from jax.experimental import pallas as pl
from jax.experimental import layout as jex_layout
from jax._src.layout import AutoLayout as _AUTO
from jax.experimental.pallas import tpu as pltpu
from jax.experimental.pallas import tpu_sc as plsc

_D = 512
_G = 4096
_GW = 20

_info = plsc.get_sparse_core_info()
_NC, _NS = _info.num_cores, _info.num_subcores
_NW = _NC * _NS
_GPW = _G // _NW
_NB = 4
_LOOK = 2
_NROUND = _GPW // _NB
_NL = _D // 16


def _make_gather():
    mesh = plsc.VectorSubcoreMesh(core_axis_name="c", subcore_axis_name="s")

    @functools.partial(
        pl.kernel,
        mesh=mesh,
        out_type=jax.ShapeDtypeStruct((_G, _GW, _D), jnp.float32),
        scratch_types=[
            pltpu.VMEM((_GPW, 16), jnp.int32),
            pltpu.VMEM((_GPW, 8), jnp.int32),
            pltpu.VMEM((_GW, _D), jnp.float32),
            pltpu.VMEM((_GW, _D), jnp.float32),
            pltpu.VMEM((_GW, _D), jnp.float32),
            pltpu.VMEM((_GW, _D), jnp.float32),
            pltpu.VMEM((8, _D), jnp.float32),
            pltpu.VMEM((8, _D), jnp.float32),
            pltpu.VMEM((8, _D), jnp.float32),
            pltpu.VMEM((8, _D), jnp.float32),
            pltpu.SemaphoreType.DMA,
            pltpu.SemaphoreType.DMA,
            pltpu.SemaphoreType.DMA,
            pltpu.SemaphoreType.DMA,
            pltpu.SemaphoreType.DMA,
            pltpu.SemaphoreType.DMA,
            pltpu.SemaphoreType.DMA,
            pltpu.SemaphoreType.DMA,
        ],
    )
    def gather_k(idxa_hbm, idxb_hbm, table_hbm, out_hbm, idxa_v, idxb_v,
                 st0, st1, st2, st3, t0, t1, t2, t3,
                 g0, g1, g2, g3, w0, w1, w2, w3):
        stage = [st0, st1, st2, st3]
        tail = [t0, t1, t2, t3]
        gsem = [g0, g1, g2, g3]
        wsem = [w0, w1, w2, w3]

        wid = lax.axis_index("s") * _NC + lax.axis_index("c")
        gbase = wid * _GPW
        pltpu.sync_copy(idxa_hbm.at[pl.ds(gbase, _GPW)], idxa_v)
        pltpu.sync_copy(idxb_hbm.at[pl.ds(gbase, _GPW)], idxb_v)

        def start_gather(c, b):
            pltpu.async_copy(table_hbm.at[idxa_v.at[c]],
                             stage[b].at[pl.ds(0, 16)], gsem[b])
            pltpu.async_copy(table_hbm.at[idxb_v.at[c]], tail[b], gsem[b])

        def wait_gather(c, b):
            pltpu.make_async_copy(table_hbm.at[idxa_v.at[c]],
                                  stage[b].at[pl.ds(0, 16)], gsem[b]).wait()
            pltpu.make_async_copy(table_hbm.at[idxb_v.at[c]], tail[b],
                                  gsem[b]).wait()

        def repack(b):
            for r in range(4):
                for l in range(_NL):
                    stage[b][16 + r, pl.ds(l * 16, 16)] = (
                        tail[b][r, pl.ds(l * 16, 16)])

        def start_wb(c, b):
            pltpu.async_copy(stage[b], out_hbm.at[gbase + c], wsem[b])

        def wait_wb(c, b):
            pltpu.make_async_copy(stage[b], out_hbm.at[gbase + c],
                                  wsem[b]).wait()

        start_gather(0, 0)
        start_gather(1, 1)

        for b in range(_NB):
            wait_gather(b, b)
            repack(b)
            start_wb(b, b)
            cn = b + _LOOK
            bn = cn % _NB
            if b >= _LOOK:
                wait_wb(cn - _NB, bn)
            start_gather(cn, bn)

        def round_body(p, carry):
            for b in range(_NB):
                c = _NB * p + b
                cn = c + _LOOK
                bn = (b + _LOOK) % _NB
                wait_gather(c, b)
                repack(b)
                start_wb(c, b)
                wait_wb(cn - _NB, bn)
                start_gather(cn, bn)
            return carry

        lax.fori_loop(1, _NROUND - 1, round_body, 0)

        for b in range(_NB):
            c = _NB * (_NROUND - 1) + b
            cn = c + _LOOK
            bn = (b + _LOOK) % _NB
            wait_gather(c, b)
            repack(b)
            start_wb(c, b)
            if cn < _GPW:
                wait_wb(cn - _NB, bn)
                start_gather(cn, bn)

        for b in range(_NB):
            wait_wb(_GPW - _NB + b, b)

    return gather_k


_gather = _make_gather()


def _kernel_impl(indices, weight):
    idx = indices.astype(jnp.int32)
    idxa = idx[:, :16]
    idxb = jnp.concatenate([idx[:, 16:], idx[:, 16:]], axis=1)
    return _gather(idxa, idxb, weight)


kernel = jax.jit(_kernel_impl)


# speedup vs baseline: 1.1063x; 1.1063x over previous
import functools

import jax
import jax.numpy as jnp
from jax import lax
from jax.experimental import pallas as pl
from jax.experimental.pallas import tpu as pltpu
from jax.experimental.pallas import tpu_sc as plsc

_D = 512
_G = 4096
_GW = 20
_GP = 24

_info = plsc.get_sparse_core_info()
_NC, _NS = _info.num_cores, _info.num_subcores
_NW = _NC * _NS
_GPW = _G // _NW
_CPG = 2
_RPC = _CPG * _GP
_NCHUNK = _GPW // _CPG
_NB = 4
_LOOK = 2
_NROUND = _NCHUNK // _NB


def _make_gather():
    mesh = plsc.VectorSubcoreMesh(core_axis_name="c", subcore_axis_name="s")

    @functools.partial(
        pl.kernel,
        mesh=mesh,
        out_type=jax.ShapeDtypeStruct((_G * _GP, _D), jnp.float32),
        scratch_types=[
            pltpu.VMEM((_NCHUNK, _RPC), jnp.int32),
            pltpu.VMEM((_RPC, _D), jnp.float32),
            pltpu.VMEM((_RPC, _D), jnp.float32),
            pltpu.VMEM((_RPC, _D), jnp.float32),
            pltpu.VMEM((_RPC, _D), jnp.float32),
            pltpu.SemaphoreType.DMA,
            pltpu.SemaphoreType.DMA,
            pltpu.SemaphoreType.DMA,
            pltpu.SemaphoreType.DMA,
            pltpu.SemaphoreType.DMA,
            pltpu.SemaphoreType.DMA,
            pltpu.SemaphoreType.DMA,
            pltpu.SemaphoreType.DMA,
        ],
    )
    def gather_k(idx_hbm, table_hbm, out_hbm, idx_v,
                 b0, b1, b2, b3, g0, g1, g2, g3, w0, w1, w2, w3):
        buf = [b0, b1, b2, b3]
        gsem = [g0, g1, g2, g3]
        wsem = [w0, w1, w2, w3]

        wid = lax.axis_index("s") * _NC + lax.axis_index("c")
        rbase = wid * _NCHUNK * _RPC
        pltpu.sync_copy(idx_hbm.at[pl.ds(wid * _NCHUNK, _NCHUNK)], idx_v)

        def start_gather(c, b):
            pltpu.async_copy(table_hbm.at[idx_v.at[c]], buf[b], gsem[b])

        def wait_gather(c, b):
            pltpu.make_async_copy(table_hbm.at[idx_v.at[c]], buf[b],
                                  gsem[b]).wait()

        def start_wb(c, b):
            pltpu.async_copy(buf[b], out_hbm.at[pl.ds(rbase + c * _RPC, _RPC)],
                             wsem[b])

        def wait_wb(c, b):
            pltpu.make_async_copy(buf[b],
                                  out_hbm.at[pl.ds(rbase + c * _RPC, _RPC)],
                                  wsem[b]).wait()

        start_gather(0, 0)
        start_gather(1, 1)

        for b in range(_NB):
            wait_gather(b, b)
            start_wb(b, b)
            cn = b + _LOOK
            bn = cn % _NB
            if b >= _LOOK:
                wait_wb(cn - _NB, bn)
            start_gather(cn, bn)

        def round_body(p, carry):
            for b in range(_NB):
                c = _NB * p + b
                cn = c + _LOOK
                bn = (b + _LOOK) % _NB
                wait_gather(c, b)
                start_wb(c, b)
                wait_wb(cn - _NB, bn)
                start_gather(cn, bn)
            return carry

        lax.fori_loop(1, _NROUND - 1, round_body, 0)

        for b in range(_NB):
            c = _NB * (_NROUND - 1) + b
            cn = c + _LOOK
            bn = (b + _LOOK) % _NB
            wait_gather(c, b)
            start_wb(c, b)
            if cn < _NCHUNK:
                wait_wb(cn - _NB, bn)
                start_gather(cn, bn)

        for b in range(_NB):
            wait_wb(_NCHUNK - _NB + b, b)

    return gather_k


_gather = _make_gather()


@jax.jit
def kernel(indices, weight):
    idx = indices.astype(jnp.int32)
    idx24 = jnp.concatenate([idx, idx[:, _GW - (_GP - _GW):]], axis=1)
    idx_chunks = idx24.reshape(_NW * _NCHUNK, _RPC)
    out = _gather(idx_chunks, weight)
    return out.reshape(_G, _GP, _D)[:, :_GW, :]
